# baseline (speedup 1.0000x reference)
import functools
import jax
import jax.numpy as jnp
from jax import lax
from jax.experimental import pallas as pl
from jax.experimental.pallas import tpu as pltpu
from jax.experimental.pallas import tpu_sc as plsc

B, L, D = 1024, 512, 128
RG = 2
CHUNK_ROWS = L // RG
NC, NS = 2, 16
NW = NC * NS
BGROUPS = NW // RG
BPW = B // BGROUPS
NBUF = 2
PD = 1
NG = (BPW - 1) // NBUF

_mesh = plsc.VectorSubcoreMesh(core_axis_name="c", subcore_axis_name="s")


@functools.partial(
    pl.kernel,
    mesh=_mesh,
    out_type=jax.ShapeDtypeStruct((B * RG, CHUNK_ROWS, D), jnp.float32),
    scratch_types=(
        [pltpu.VMEM((CHUNK_ROWS, D), jnp.float32)]
        + [pltpu.VMEM((CHUNK_ROWS, D), jnp.float32) for _ in range(NBUF)]
        + [pltpu.SemaphoreType.DMA for _ in range(2 * NBUF)]
    ),
)
def _sc_add(embed_hbm, pos_hbm, out_hbm, pos_v, *rest):
    bufs = rest[:NBUF]
    in_sems = rest[NBUF:2 * NBUF]
    out_sems = rest[2 * NBUF:]

    wid = lax.axis_index("s") * NC + lax.axis_index("c")
    bg = wid // RG
    rg = wid % RG
    base = bg * BPW

    def chunk_idx(k):
        return (base + k) * RG + rg

    pltpu.sync_copy(pos_hbm.at[rg], pos_v)

    for j in range(PD):
        pltpu.async_copy(embed_hbm.at[chunk_idx(j)], bufs[j], in_sems[j])

    def iteration(k, s, wait_out_pred, start_in_pred):
        buf = bufs[s]
        c = chunk_idx(k)
        pltpu.make_async_copy(embed_hbm.at[c], buf, in_sems[s]).wait()

        def add_body(r, carry2):
            for j in range(D // 16):
                sl = pl.ds(j * 16, 16)
                plsc.addupdate(buf.at[r, sl], pos_v[r, sl])
            return carry2

        lax.fori_loop(0, CHUNK_ROWS, add_body, 0)

        pltpu.async_copy(buf, out_hbm.at[c], out_sems[s])

        sp = (s + PD) % NBUF
        kw = k + PD - NBUF

        def retire():
            pltpu.make_async_copy(
                bufs[sp], out_hbm.at[chunk_idx(kw)], out_sems[sp]
            ).wait()

        if wait_out_pred is None:
            retire()
        elif wait_out_pred is not False:
            pl.when(wait_out_pred)(retire)

        def launch():
            pltpu.async_copy(
                embed_hbm.at[chunk_idx(k + PD)], bufs[sp], in_sems[sp]
            )

        if start_in_pred is None:
            launch()
        elif start_in_pred is not False:
            pl.when(start_in_pred)(launch)

    def group(g, carry):
        for s in range(NBUF):
            k = g * NBUF + s
            wait_out_pred = (g >= 1) if s < NBUF - PD else None
            if (NG - 1) * NBUF + s + PD >= BPW:
                start_in_pred = g < NG - 1
            else:
                start_in_pred = None
            iteration(k, s, wait_out_pred, start_in_pred)
        return carry

    lax.fori_loop(0, NG, group, 0)

    for k in range(NG * NBUF, BPW):
        iteration(k, k % NBUF, None, None if k + PD < BPW else False)

    for j in range(NBUF - PD):
        kk = BPW - (NBUF - PD) + j
        pltpu.make_async_copy(
            bufs[kk % NBUF], out_hbm.at[chunk_idx(kk)], out_sems[kk % NBUF]
        ).wait()


def kernel(embed, pos_table):
    e = embed.reshape(B * RG, CHUNK_ROWS, D)
    p = pos_table.reshape(RG, CHUNK_ROWS, D)
    out = _sc_add(e, p)
    return out.reshape(B, L, D)


# speedup vs baseline: 1.6198x; 1.6198x over previous
import functools
import jax
import jax.numpy as jnp
from jax import lax
from jax.experimental import pallas as pl
from jax.experimental.pallas import tpu as pltpu
from jax.experimental.pallas import tpu_sc as plsc

B, L, D = 1024, 512, 128
RG = 4
CHUNK_ROWS = L // RG
NC, NS = 2, 16
NW = NC * NS
BGROUPS = NW // RG
BPW = B // BGROUPS
NBUF = 6
PD = 4
NG = (BPW - 1) // NBUF

_mesh = plsc.VectorSubcoreMesh(core_axis_name="c", subcore_axis_name="s")


@functools.partial(
    pl.kernel,
    mesh=_mesh,
    out_type=jax.ShapeDtypeStruct((B * RG, CHUNK_ROWS, D), jnp.float32),
    scratch_types=(
        [pltpu.VMEM((CHUNK_ROWS, D), jnp.float32)]
        + [pltpu.VMEM((CHUNK_ROWS, D), jnp.float32) for _ in range(NBUF)]
        + [pltpu.SemaphoreType.DMA for _ in range(2 * NBUF)]
    ),
)
def _sc_add(embed_hbm, pos_hbm, out_hbm, pos_v, *rest):
    bufs = rest[:NBUF]
    in_sems = rest[NBUF:2 * NBUF]
    out_sems = rest[2 * NBUF:]

    wid = lax.axis_index("s") * NC + lax.axis_index("c")
    bg = wid // RG
    rg = wid % RG
    base = bg * BPW

    def chunk_idx(k):
        return (base + k) * RG + rg

    pltpu.sync_copy(pos_hbm.at[rg], pos_v)

    for j in range(PD):
        pltpu.async_copy(embed_hbm.at[chunk_idx(j)], bufs[j], in_sems[j])

    def iteration(k, s, wait_out_pred, start_in_pred):
        buf = bufs[s]
        c = chunk_idx(k)
        pltpu.make_async_copy(embed_hbm.at[c], buf, in_sems[s]).wait()

        def add_body(r, carry2):
            for j in range(D // 16):
                sl = pl.ds(j * 16, 16)
                plsc.addupdate(buf.at[r, sl], pos_v[r, sl])
            return carry2

        lax.fori_loop(0, CHUNK_ROWS, add_body, 0)

        pltpu.async_copy(buf, out_hbm.at[c], out_sems[s])

        sp = (s + PD) % NBUF
        kw = k + PD - NBUF

        def retire():
            pltpu.make_async_copy(
                bufs[sp], out_hbm.at[chunk_idx(kw)], out_sems[sp]
            ).wait()

        if wait_out_pred is None:
            retire()
        elif wait_out_pred is not False:
            pl.when(wait_out_pred)(retire)

        def launch():
            pltpu.async_copy(
                embed_hbm.at[chunk_idx(k + PD)], bufs[sp], in_sems[sp]
            )

        if start_in_pred is None:
            launch()
        elif start_in_pred is not False:
            pl.when(start_in_pred)(launch)

    def group(g, carry):
        for s in range(NBUF):
            k = g * NBUF + s
            wait_out_pred = (g >= 1) if s < NBUF - PD else None
            if (NG - 1) * NBUF + s + PD >= BPW:
                start_in_pred = g < NG - 1
            else:
                start_in_pred = None
            iteration(k, s, wait_out_pred, start_in_pred)
        return carry

    lax.fori_loop(0, NG, group, 0)

    for k in range(NG * NBUF, BPW):
        iteration(k, k % NBUF, None, None if k + PD < BPW else False)

    for j in range(NBUF - PD):
        kk = BPW - (NBUF - PD) + j
        pltpu.make_async_copy(
            bufs[kk % NBUF], out_hbm.at[chunk_idx(kk)], out_sems[kk % NBUF]
        ).wait()


def kernel(embed, pos_table):
    e = embed.reshape(B * RG, CHUNK_ROWS, D)
    p = pos_table.reshape(RG, CHUNK_ROWS, D)
    out = _sc_add(e, p)
    return out.reshape(B, L, D)


# speedup vs baseline: 1.6246x; 1.0029x over previous
import functools
import jax
import jax.numpy as jnp
from jax import lax
from jax.experimental import pallas as pl
from jax.experimental.pallas import tpu as pltpu
from jax.experimental.pallas import tpu_sc as plsc

B, L, D = 1024, 512, 128
RG = 4
CHUNK_ROWS = L // RG
NC, NS = 2, 16
NW = NC * NS
BGROUPS = NW // RG
BPW = B // BGROUPS
NBUF = 6
PD = 4
NG = (BPW - 1) // NBUF

_mesh = plsc.VectorSubcoreMesh(core_axis_name="c", subcore_axis_name="s")


@functools.partial(
    pl.kernel,
    mesh=_mesh,
    compiler_params=pltpu.CompilerParams(use_tc_tiling_on_sc=True),
    out_type=jax.ShapeDtypeStruct((B * RG, CHUNK_ROWS, D), jnp.float32),
    scratch_types=(
        [pltpu.VMEM((CHUNK_ROWS, D), jnp.float32)]
        + [pltpu.VMEM((CHUNK_ROWS, D), jnp.float32) for _ in range(NBUF)]
        + [pltpu.SemaphoreType.DMA for _ in range(2 * NBUF)]
    ),
)
def _sc_add(embed_hbm, pos_hbm, out_hbm, pos_v, *rest):
    bufs = rest[:NBUF]
    in_sems = rest[NBUF:2 * NBUF]
    out_sems = rest[2 * NBUF:]

    wid = lax.axis_index("s") * NC + lax.axis_index("c")
    bg = wid // RG
    rg = wid % RG
    base = bg * BPW

    def chunk_idx(k):
        return (base + k) * RG + rg

    pltpu.sync_copy(pos_hbm.at[rg], pos_v)

    for j in range(PD):
        pltpu.async_copy(embed_hbm.at[chunk_idx(j)], bufs[j], in_sems[j])

    def iteration(k, s, wait_out_pred, start_in_pred):
        buf = bufs[s]
        c = chunk_idx(k)
        pltpu.make_async_copy(embed_hbm.at[c], buf, in_sems[s]).wait()

        def add_body(r, carry2):
            for j in range(D // 16):
                sl = pl.ds(j * 16, 16)
                plsc.addupdate(buf.at[r, sl], pos_v[r, sl])
            return carry2

        lax.fori_loop(0, CHUNK_ROWS, add_body, 0)

        pltpu.async_copy(buf, out_hbm.at[c], out_sems[s])

        sp = (s + PD) % NBUF
        kw = k + PD - NBUF

        def retire():
            pltpu.make_async_copy(
                bufs[sp], out_hbm.at[chunk_idx(kw)], out_sems[sp]
            ).wait()

        if wait_out_pred is None:
            retire()
        elif wait_out_pred is not False:
            pl.when(wait_out_pred)(retire)

        def launch():
            pltpu.async_copy(
                embed_hbm.at[chunk_idx(k + PD)], bufs[sp], in_sems[sp]
            )

        if start_in_pred is None:
            launch()
        elif start_in_pred is not False:
            pl.when(start_in_pred)(launch)

    def group(g, carry):
        for s in range(NBUF):
            k = g * NBUF + s
            wait_out_pred = (g >= 1) if s < NBUF - PD else None
            if (NG - 1) * NBUF + s + PD >= BPW:
                start_in_pred = g < NG - 1
            else:
                start_in_pred = None
            iteration(k, s, wait_out_pred, start_in_pred)
        return carry

    lax.fori_loop(0, NG, group, 0)

    for k in range(NG * NBUF, BPW):
        iteration(k, k % NBUF, None, None if k + PD < BPW else False)

    for j in range(NBUF - PD):
        kk = BPW - (NBUF - PD) + j
        pltpu.make_async_copy(
            bufs[kk % NBUF], out_hbm.at[chunk_idx(kk)], out_sems[kk % NBUF]
        ).wait()


def kernel(embed, pos_table):
    e = embed.reshape(B * RG, CHUNK_ROWS, D)
    p = pos_table.reshape(RG, CHUNK_ROWS, D)
    out = _sc_add(e, p)
    return out.reshape(B, L, D)
